# baseline (speedup 1.0000x reference)
import functools

import jax
import jax.numpy as jnp
from jax import lax
from jax.experimental import pallas as pl
from jax.experimental.pallas import tpu as pltpu
from jax.experimental.pallas import tpu_sc as plsc

N = 10000
E = 160000
G = 64
NF = 128
EF = 16
NG = 16
OC = 32

NW = 32
CH = 128
E_PAD = 163840
CPW = E_PAD // (NW * CH)
N_ACC = N + CH
EB = 640
NB = 1000

_mesh = lambda: plsc.VectorSubcoreMesh(core_axis_name="c", subcore_axis_name="s")


def _sc_gather2(xb, src, dst):
    @functools.partial(
        pl.kernel,
        mesh=_mesh(),
        out_type=[jax.ShapeDtypeStruct((E_PAD, NF), jnp.float32),
                  jax.ShapeDtypeStruct((E_PAD, NF), jnp.float32)],
        scratch_types=[pltpu.VMEM((CPW * CH,), jnp.int32),
                       pltpu.VMEM((CPW * CH,), jnp.int32),
                       pltpu.VMEM((CH, NF), jnp.float32),
                       pltpu.VMEM((CH, NF), jnp.float32),
                       pltpu.VMEM_SHARED((N, NF), jnp.float32),
                       pltpu.SemaphoreType.DMA,
                       pltpu.SemaphoreType.DMA],
    )
    def k(x_hbm, src_hbm, dst_hbm, xs_hbm, xd_hbm, si_v, di_v, sb_v, db_v,
          x_sh, gsem, ssem):
        cid = lax.axis_index("c")
        sid = lax.axis_index("s")
        wid = sid * 2 + cid
        base = wid * (CPW * CH)
        @pl.when(sid < 15)
        def _():
            pltpu.sync_copy(x_hbm.at[pl.ds(sid * 632, 632)],
                            x_sh.at[pl.ds(sid * 632, 632)])

        @pl.when(sid == 15)
        def _():
            pltpu.sync_copy(x_hbm.at[pl.ds(15 * 632, N - 15 * 632)],
                            x_sh.at[pl.ds(15 * 632, N - 15 * 632)])

        pltpu.sync_copy(src_hbm.at[pl.ds(base, CPW * CH)], si_v)
        pltpu.sync_copy(dst_hbm.at[pl.ds(base, CPW * CH)], di_v)
        plsc.subcore_barrier()

        def body(j, _):
            off = j * CH
            cs = pltpu.async_copy(x_sh.at[si_v.at[pl.ds(off, CH)]], sb_v,
                                  gsem)
            cd = pltpu.async_copy(x_sh.at[di_v.at[pl.ds(off, CH)]], db_v,
                                  gsem)
            cs.wait()
            cd.wait()
            s1 = pltpu.async_copy(sb_v, xs_hbm.at[pl.ds(base + off, CH)],
                                  ssem)
            s2 = pltpu.async_copy(db_v, xd_hbm.at[pl.ds(base + off, CH)],
                                  ssem)
            s1.wait()
            s2.wait()
            return 0

        lax.fori_loop(0, CPW, body, 0)

    return k(xb, src, dst)


def _sc_scatter(p, dst2d, zeros):
    ZR = 632
    WR = 624

    @functools.partial(
        pl.kernel,
        mesh=_mesh(),
        out_type=jax.ShapeDtypeStruct((2 * N, NF), jnp.float32),
        scratch_types=[pltpu.VMEM((CPW, CH), jnp.int32),
                       pltpu.VMEM((CH, NF), jnp.float32),
                       pltpu.VMEM((CH, NF), jnp.float32),
                       pltpu.VMEM_SHARED((N_ACC, NF), jnp.float32),
                       pltpu.SemaphoreType.DMA,
                       pltpu.SemaphoreType.DMA],
    )
    def k(p_hbm, dst_hbm, z_hbm, out_hbm, idx_v, rows_a, rows_b, acc_sh,
          sem_a, sem_b):
        cid = lax.axis_index("c")
        sid = lax.axis_index("s")
        t = cid * 16 + sid
        pltpu.sync_copy(z_hbm.at[pl.ds(sid * ZR, ZR)],
                        acc_sh.at[pl.ds(sid * ZR, ZR)])
        plsc.subcore_barrier()
        pltpu.sync_copy(dst_hbm.at[pl.ds(t * CPW, CPW)], idx_v)
        eb = t * CPW * CH
        pltpu.async_copy(p_hbm.at[pl.ds(eb, CH)], rows_a, sem_a)

        def body(gg, _):
            j0 = 2 * gg
            pltpu.async_copy(p_hbm.at[pl.ds(eb + (j0 + 1) * CH, CH)],
                             rows_b, sem_b)
            pltpu.make_async_copy(p_hbm.at[pl.ds(eb, CH)], rows_a,
                                  sem_a).wait()
            pltpu.sync_copy(rows_a, acc_sh.at[idx_v.at[j0]], add=True)

            @pl.when(j0 + 2 < CPW)
            def _():
                pltpu.async_copy(p_hbm.at[pl.ds(eb + (j0 + 2) * CH, CH)],
                                 rows_a, sem_a)

            pltpu.make_async_copy(p_hbm.at[pl.ds(eb, CH)], rows_b,
                                  sem_b).wait()
            pltpu.sync_copy(rows_b, acc_sh.at[idx_v.at[j0 + 1]], add=True)
            return 0

        lax.fori_loop(0, CPW // 2, body, 0)
        plsc.subcore_barrier()
        pltpu.sync_copy(acc_sh.at[pl.ds(sid * WR, WR)],
                        out_hbm.at[pl.ds(cid * N + sid * WR, WR)])

        @pl.when(sid == 15)
        def _():
            pltpu.sync_copy(acc_sh.at[pl.ds(16 * WR, N - 16 * WR)],
                            out_hbm.at[pl.ds(cid * N + 16 * WR, N - 16 * WR)])

    return k(p, dst2d, zeros)


def _sc_count(dst2d, zeros16, ones16):
    ZR = 632
    WR = 624

    @functools.partial(
        pl.kernel,
        mesh=_mesh(),
        out_type=jax.ShapeDtypeStruct((2 * N, NF), jnp.float32),
        scratch_types=[pltpu.VMEM((CPW, CH), jnp.int32),
                       pltpu.VMEM((CH, NF), jnp.float32),
                       pltpu.VMEM_SHARED((N_ACC, NF), jnp.float32)],
    )
    def k(dst_hbm, z_hbm, ones_hbm, out_hbm, idx_v, ones_v, acc_sh):
        cid = lax.axis_index("c")
        sid = lax.axis_index("s")
        t = cid * 16 + sid
        pltpu.sync_copy(z_hbm.at[pl.ds(sid * ZR, ZR)],
                        acc_sh.at[pl.ds(sid * ZR, ZR)])
        plsc.subcore_barrier()
        pltpu.sync_copy(ones_hbm, ones_v)
        pltpu.sync_copy(dst_hbm.at[pl.ds(t * CPW, CPW)], idx_v)

        def body(j, _):
            pltpu.sync_copy(ones_v, acc_sh.at[idx_v.at[j]], add=True)
            return 0

        lax.fori_loop(0, CPW, body, 0)
        plsc.subcore_barrier()
        pltpu.sync_copy(acc_sh.at[pl.ds(sid * WR, WR)],
                        out_hbm.at[pl.ds(cid * N + sid * WR, WR)])

        @pl.when(sid == 15)
        def _():
            pltpu.sync_copy(acc_sh.at[pl.ds(16 * WR, N - 16 * WR)],
                            out_hbm.at[pl.ds(cid * N + 16 * WR, N - 16 * WR)])

    return k(dst2d, zeros16, ones16)


def _edge_layer(xs, xd, e, We_s, We_d, We_e, be, Wn_a, write_eout):
    K = e.shape[1]
    EO = We_s.shape[1]
    grid = E_PAD // EB

    def f(xs_ref, xd_ref, e_ref, ws_ref, wd_ref, we_ref, be_ref, wa_ref,
          *out_refs):
        acc = jnp.dot(xs_ref[...].astype(jnp.bfloat16), ws_ref[...],
                      preferred_element_type=jnp.float32)
        acc += jnp.dot(xd_ref[...].astype(jnp.bfloat16), wd_ref[...],
                       preferred_element_type=jnp.float32)
        acc += jnp.dot(e_ref[...], we_ref[...],
                       preferred_element_type=jnp.float32)
        acc += be_ref[...]
        r = jnp.maximum(acc, 0.0)
        if write_eout:
            out_refs[1][...] = r.astype(jnp.bfloat16)
        out_refs[0][...] = jnp.dot(r, wa_ref[...],
                                   preferred_element_type=jnp.float32)

    out_shape = [jax.ShapeDtypeStruct((E_PAD, NF), jnp.float32)]
    out_specs = [pl.BlockSpec((EB, NF), lambda i: (i, 0))]
    if write_eout:
        out_shape.append(jax.ShapeDtypeStruct((E_PAD, EO), jnp.bfloat16))
        out_specs.append(pl.BlockSpec((EB, EO), lambda i: (i, 0)))

    res = pl.pallas_call(
        f,
        grid=(grid,),
        in_specs=[pl.BlockSpec((EB, NF), lambda i: (i, 0)),
                  pl.BlockSpec((EB, NF), lambda i: (i, 0)),
                  pl.BlockSpec((EB, K), lambda i: (i, 0)),
                  pl.BlockSpec((NF, EO), lambda i: (0, 0)),
                  pl.BlockSpec((NF, EO), lambda i: (0, 0)),
                  pl.BlockSpec((K, EO), lambda i: (0, 0)),
                  pl.BlockSpec((1, EO), lambda i: (0, 0)),
                  pl.BlockSpec((EO, NF), lambda i: (0, 0))],
        out_specs=out_specs,
        out_shape=out_shape,
    )(xs, xd, e, We_s, We_d, We_e, be.reshape(1, -1), Wn_a)
    if write_eout:
        return res[1], res[0]
    return None, res[0]


def _node_layer(x, ap, cntp, batch3, u, Wn_x, bn, Wg_g, Wg_u, bg):
    GO = Wg_g.shape[1]
    GU = u.shape[1]
    nblk = N // NB

    def f(x_ref, ap_ref, cnt_ref, b_ref, u_ref, wx_ref, bn_ref, wg_ref,
          wu_ref, bg_ref, xo_ref, uo_ref, gacc, gcnt):
        i = pl.program_id(0)

        @pl.when(i == 0)
        def _():
            gacc[...] = jnp.zeros_like(gacc)
            gcnt[...] = jnp.zeros_like(gcnt)

        A = ap_ref[0] + ap_ref[1]
        c = cnt_ref[0][:, 0:1] + cnt_ref[1][:, 0:1]
        inv = 1.0 / jnp.maximum(c, 1.0)
        acc = jnp.dot(x_ref[...], wx_ref[...],
                      preferred_element_type=jnp.float32)
        acc += A * inv + bn_ref[...]
        xo = jnp.maximum(acc, 0.0)
        xo_ref[...] = xo
        b = b_ref[0]
        gids = lax.broadcasted_iota(jnp.int32, (G, NB), 0)
        S = (b == gids).astype(jnp.float32)
        gacc[...] += jnp.dot(S, xo, preferred_element_type=jnp.float32)
        gcnt[...] += jnp.broadcast_to(
            jnp.sum(S, axis=1, keepdims=True), gcnt.shape)

        @pl.when(i == nblk - 1)
        def _():
            gm = gacc[...] / jnp.maximum(gcnt[...], 1.0)
            uo = jnp.dot(gm, wg_ref[...], preferred_element_type=jnp.float32)
            uo += jnp.dot(u_ref[...], wu_ref[...],
                          preferred_element_type=jnp.float32)
            uo += bg_ref[...]
            uo_ref[...] = jnp.maximum(uo, 0.0)

    return pl.pallas_call(
        f,
        grid=(nblk,),
        in_specs=[pl.BlockSpec((NB, NF), lambda i: (i, 0)),
                  pl.BlockSpec((2, NB, NF), lambda i: (0, i, 0)),
                  pl.BlockSpec((2, NB, NF), lambda i: (0, i, 0)),
                  pl.BlockSpec((1, 1, NB), lambda i: (i, 0, 0)),
                  pl.BlockSpec((G, GU), lambda i: (0, 0)),
                  pl.BlockSpec((NF, NF), lambda i: (0, 0)),
                  pl.BlockSpec((1, NF), lambda i: (0, 0)),
                  pl.BlockSpec((NF, GO), lambda i: (0, 0)),
                  pl.BlockSpec((GU, GO), lambda i: (0, 0)),
                  pl.BlockSpec((1, GO), lambda i: (0, 0))],
        out_specs=[pl.BlockSpec((NB, NF), lambda i: (i, 0)),
                   pl.BlockSpec((G, GO), lambda i: (0, 0))],
        out_shape=[jax.ShapeDtypeStruct((N, NF), jnp.float32),
                   jax.ShapeDtypeStruct((G, GO), jnp.float32)],
        scratch_shapes=[pltpu.VMEM((G, NF), jnp.float32),
                        pltpu.VMEM((G, NF), jnp.float32)],
        compiler_params=pltpu.CompilerParams(
            dimension_semantics=("arbitrary",)),
    )(x, ap, cntp, batch3, u, Wn_x, bn.reshape(1, -1), Wg_g, Wg_u,
      bg.reshape(1, -1))


def _final(rgo, pgo, W1r, W1p, b1, W2pad, b2pad):
    def f(r_ref, p_ref, w1r_ref, w1p_ref, b1_ref, w2_ref, b2_ref, o_ref):
        y = jnp.dot(r_ref[...], w1r_ref[...],
                    preferred_element_type=jnp.float32)
        y += jnp.dot(p_ref[...], w1p_ref[...],
                     preferred_element_type=jnp.float32)
        y = jnp.maximum(y + b1_ref[...], 0.0)
        z = jnp.dot(y, w2_ref[...], preferred_element_type=jnp.float32)
        z += b2_ref[...]
        lane = lax.broadcasted_iota(jnp.int32, z.shape, 1)
        z = jnp.where(lane < OC, z, -1e30)
        m = jnp.max(z, axis=1, keepdims=True)
        s = jnp.sum(jnp.exp(z - m), axis=1, keepdims=True)
        o_ref[...] = z - m - jnp.log(s)

    return pl.pallas_call(
        f,
        out_shape=jax.ShapeDtypeStruct((G, 128), jnp.float32),
    )(rgo, pgo, W1r, W1p, b1.reshape(1, -1), W2pad, b2pad.reshape(1, -1))


class _GraphState:
    def __init__(self, x, e, ci, b, u, prefix):
        self.x, self.u, self.prefix = x, u, prefix
        self.src = jnp.pad(ci[0], (0, E_PAD - E))
        self.dst_g = jnp.pad(ci[1], (0, E_PAD - E))
        self.dst2d = jnp.pad(ci[1], (0, E_PAD - E),
                             constant_values=N).reshape(E_PAD // CH, CH)
        self.e = jnp.pad(e, ((0, E_PAD - E), (0, 0))).astype(jnp.bfloat16)
        self.batch3 = b.reshape(N // NB, 1, NB)


def _run_graphs(ga, gb, params):
    zeros = jnp.zeros((N_ACC, NF), jnp.float32)
    ones128 = jnp.ones((CH, NF), jnp.float32)
    for g in (ga, gb):
        g.cntp = _sc_count(g.dst2d, zeros, ones128).reshape(2, N, NF)
    for i in range(4):
        for g in (ga, gb):
            p = lambda n: params[g.prefix + str(i) + "_" + n]
            g.We, g.be = p("We"), p("be")
            g.Wn, g.bn = p("Wn"), p("bn")
            g.Wg, g.bg = p("Wg"), p("bg")
        for g in (ga, gb):
            g.xs, g.xd = _sc_gather2(g.x, g.src, g.dst_g)
        for g in (ga, gb):
            bf = jnp.bfloat16
            g.e, g.pm = _edge_layer(g.xs, g.xd, g.e, g.We[:NF].astype(bf),
                                    g.We[NF:2 * NF].astype(bf),
                                    g.We[2 * NF:].astype(bf), g.be,
                                    g.Wn[NF:], write_eout=(i < 3))
        for g in (ga, gb):
            g.ap = _sc_scatter(g.pm, g.dst2d, zeros).reshape(2, N, NF)
        for g in (ga, gb):
            g.x, g.u = _node_layer(g.x, g.ap, g.cntp, g.batch3, g.u,
                                   g.Wn[:NF], g.bn, g.Wg[:NF], g.Wg[NF:],
                                   g.bg)
    return ga.u, gb.u


def kernel(rx, re, rc, rb, rg, px, pe, pc, pb, pg, params):
    gr = _GraphState(rx, re, rc, rb, rg, "r")
    gp = _GraphState(px, pe, pc, pb, pg, "p")
    rgo, pgo = _run_graphs(gr, gp, params)
    W1, b1 = params["W1"], params["b1"]
    W2, b2 = params["W2"], params["b2"]
    W2pad = jnp.pad(W2, ((0, 0), (0, 128 - OC)))
    b2pad = jnp.pad(b2, (0, 128 - OC))
    out = _final(rgo, pgo, W1[:256], W1[256:], b1, W2pad, b2pad)
    return out[:, :OC]


# speedup vs baseline: 1.0709x; 1.0709x over previous
import functools

import jax
import jax.numpy as jnp
from jax import lax
from jax.experimental import pallas as pl
from jax.experimental.pallas import tpu as pltpu
from jax.experimental.pallas import tpu_sc as plsc

N = 10000
E = 160000
G = 64
NF = 128
EF = 16
NG = 16
OC = 32

NW = 32
CH = 128
E_PAD = 163840
CPW = E_PAD // (NW * CH)
N_ACC = N + CH
EB = 1280
NB = 2000

_mesh = lambda: plsc.VectorSubcoreMesh(core_axis_name="c", subcore_axis_name="s")


def _sc_gather2(xb, src, dst):
    @functools.partial(
        pl.kernel,
        mesh=_mesh(),
        out_type=[jax.ShapeDtypeStruct((E_PAD, NF), jnp.float32),
                  jax.ShapeDtypeStruct((E_PAD, NF), jnp.float32)],
        scratch_types=[pltpu.VMEM((CPW * CH,), jnp.int32),
                       pltpu.VMEM((CPW * CH,), jnp.int32),
                       pltpu.VMEM((CH, NF), jnp.float32),
                       pltpu.VMEM((CH, NF), jnp.float32),
                       pltpu.VMEM_SHARED((N, NF), jnp.float32),
                       pltpu.SemaphoreType.DMA,
                       pltpu.SemaphoreType.DMA],
    )
    def k(x_hbm, src_hbm, dst_hbm, xs_hbm, xd_hbm, si_v, di_v, sb_v, db_v,
          x_sh, gsem, ssem):
        cid = lax.axis_index("c")
        sid = lax.axis_index("s")
        wid = sid * 2 + cid
        base = wid * (CPW * CH)
        @pl.when(sid < 15)
        def _():
            pltpu.sync_copy(x_hbm.at[pl.ds(sid * 632, 632)],
                            x_sh.at[pl.ds(sid * 632, 632)])

        @pl.when(sid == 15)
        def _():
            pltpu.sync_copy(x_hbm.at[pl.ds(15 * 632, N - 15 * 632)],
                            x_sh.at[pl.ds(15 * 632, N - 15 * 632)])

        pltpu.sync_copy(src_hbm.at[pl.ds(base, CPW * CH)], si_v)
        pltpu.sync_copy(dst_hbm.at[pl.ds(base, CPW * CH)], di_v)
        plsc.subcore_barrier()

        def body(j, _):
            off = j * CH
            cs = pltpu.async_copy(x_sh.at[si_v.at[pl.ds(off, CH)]], sb_v,
                                  gsem)
            cd = pltpu.async_copy(x_sh.at[di_v.at[pl.ds(off, CH)]], db_v,
                                  gsem)
            cs.wait()
            cd.wait()
            s1 = pltpu.async_copy(sb_v, xs_hbm.at[pl.ds(base + off, CH)],
                                  ssem)
            s2 = pltpu.async_copy(db_v, xd_hbm.at[pl.ds(base + off, CH)],
                                  ssem)
            s1.wait()
            s2.wait()
            return 0

        lax.fori_loop(0, CPW, body, 0)

    return k(xb, src, dst)


def _sc_scatter(p, dst2d, zeros):
    ZR = 632
    WR = 624

    @functools.partial(
        pl.kernel,
        mesh=_mesh(),
        out_type=jax.ShapeDtypeStruct((2 * N, NF), jnp.float32),
        scratch_types=[pltpu.VMEM((CPW, CH), jnp.int32),
                       pltpu.VMEM((CH, NF), jnp.float32),
                       pltpu.VMEM((CH, NF), jnp.float32),
                       pltpu.VMEM_SHARED((N_ACC, NF), jnp.float32),
                       pltpu.SemaphoreType.DMA,
                       pltpu.SemaphoreType.DMA],
    )
    def k(p_hbm, dst_hbm, z_hbm, out_hbm, idx_v, rows_a, rows_b, acc_sh,
          sem_a, sem_b):
        cid = lax.axis_index("c")
        sid = lax.axis_index("s")
        t = cid * 16 + sid
        pltpu.sync_copy(z_hbm.at[pl.ds(sid * ZR, ZR)],
                        acc_sh.at[pl.ds(sid * ZR, ZR)])
        plsc.subcore_barrier()
        pltpu.sync_copy(dst_hbm.at[pl.ds(t * CPW, CPW)], idx_v)
        eb = t * CPW * CH
        pltpu.async_copy(p_hbm.at[pl.ds(eb, CH)], rows_a, sem_a)

        def body(gg, _):
            j0 = 2 * gg
            pltpu.async_copy(p_hbm.at[pl.ds(eb + (j0 + 1) * CH, CH)],
                             rows_b, sem_b)
            pltpu.make_async_copy(p_hbm.at[pl.ds(eb, CH)], rows_a,
                                  sem_a).wait()
            pltpu.sync_copy(rows_a, acc_sh.at[idx_v.at[j0]], add=True)

            @pl.when(j0 + 2 < CPW)
            def _():
                pltpu.async_copy(p_hbm.at[pl.ds(eb + (j0 + 2) * CH, CH)],
                                 rows_a, sem_a)

            pltpu.make_async_copy(p_hbm.at[pl.ds(eb, CH)], rows_b,
                                  sem_b).wait()
            pltpu.sync_copy(rows_b, acc_sh.at[idx_v.at[j0 + 1]], add=True)
            return 0

        lax.fori_loop(0, CPW // 2, body, 0)
        plsc.subcore_barrier()
        pltpu.sync_copy(acc_sh.at[pl.ds(sid * WR, WR)],
                        out_hbm.at[pl.ds(cid * N + sid * WR, WR)])

        @pl.when(sid == 15)
        def _():
            pltpu.sync_copy(acc_sh.at[pl.ds(16 * WR, N - 16 * WR)],
                            out_hbm.at[pl.ds(cid * N + 16 * WR, N - 16 * WR)])

    return k(p, dst2d, zeros)


def _sc_count(dst2d, zeros16, ones16):
    ZR = 632
    WR = 624

    @functools.partial(
        pl.kernel,
        mesh=_mesh(),
        out_type=jax.ShapeDtypeStruct((2 * N, NF), jnp.float32),
        scratch_types=[pltpu.VMEM((CPW, CH), jnp.int32),
                       pltpu.VMEM((CH, NF), jnp.float32),
                       pltpu.VMEM_SHARED((N_ACC, NF), jnp.float32)],
    )
    def k(dst_hbm, z_hbm, ones_hbm, out_hbm, idx_v, ones_v, acc_sh):
        cid = lax.axis_index("c")
        sid = lax.axis_index("s")
        t = cid * 16 + sid
        pltpu.sync_copy(z_hbm.at[pl.ds(sid * ZR, ZR)],
                        acc_sh.at[pl.ds(sid * ZR, ZR)])
        plsc.subcore_barrier()
        pltpu.sync_copy(ones_hbm, ones_v)
        pltpu.sync_copy(dst_hbm.at[pl.ds(t * CPW, CPW)], idx_v)

        def body(j, _):
            pltpu.sync_copy(ones_v, acc_sh.at[idx_v.at[j]], add=True)
            return 0

        lax.fori_loop(0, CPW, body, 0)
        plsc.subcore_barrier()
        pltpu.sync_copy(acc_sh.at[pl.ds(sid * WR, WR)],
                        out_hbm.at[pl.ds(cid * N + sid * WR, WR)])

        @pl.when(sid == 15)
        def _():
            pltpu.sync_copy(acc_sh.at[pl.ds(16 * WR, N - 16 * WR)],
                            out_hbm.at[pl.ds(cid * N + 16 * WR, N - 16 * WR)])

    return k(dst2d, zeros16, ones16)


def _edge_layer(xs, xd, e, We_s, We_d, We_e, be, Wn_a, write_eout):
    K = e.shape[1]
    EO = We_s.shape[1]
    grid = E_PAD // EB

    def f(xs_ref, xd_ref, e_ref, ws_ref, wd_ref, we_ref, be_ref, wa_ref,
          *out_refs):
        acc = jnp.dot(xs_ref[...].astype(jnp.bfloat16), ws_ref[...],
                      preferred_element_type=jnp.float32)
        acc += jnp.dot(xd_ref[...].astype(jnp.bfloat16), wd_ref[...],
                       preferred_element_type=jnp.float32)
        acc += jnp.dot(e_ref[...], we_ref[...],
                       preferred_element_type=jnp.float32)
        acc += be_ref[...]
        r = jnp.maximum(acc, 0.0)
        if write_eout:
            out_refs[1][...] = r.astype(jnp.bfloat16)
        out_refs[0][...] = jnp.dot(r, wa_ref[...],
                                   preferred_element_type=jnp.float32)

    out_shape = [jax.ShapeDtypeStruct((E_PAD, NF), jnp.float32)]
    out_specs = [pl.BlockSpec((EB, NF), lambda i: (i, 0))]
    if write_eout:
        out_shape.append(jax.ShapeDtypeStruct((E_PAD, EO), jnp.bfloat16))
        out_specs.append(pl.BlockSpec((EB, EO), lambda i: (i, 0)))

    res = pl.pallas_call(
        f,
        grid=(grid,),
        in_specs=[pl.BlockSpec((EB, NF), lambda i: (i, 0)),
                  pl.BlockSpec((EB, NF), lambda i: (i, 0)),
                  pl.BlockSpec((EB, K), lambda i: (i, 0)),
                  pl.BlockSpec((NF, EO), lambda i: (0, 0)),
                  pl.BlockSpec((NF, EO), lambda i: (0, 0)),
                  pl.BlockSpec((K, EO), lambda i: (0, 0)),
                  pl.BlockSpec((1, EO), lambda i: (0, 0)),
                  pl.BlockSpec((EO, NF), lambda i: (0, 0))],
        out_specs=out_specs,
        out_shape=out_shape,
    )(xs, xd, e, We_s, We_d, We_e, be.reshape(1, -1), Wn_a)
    if write_eout:
        return res[1], res[0]
    return None, res[0]


def _node_layer(x, ap, cntp, batch3, u, Wn_x, bn, Wg_g, Wg_u, bg):
    GO = Wg_g.shape[1]
    GU = u.shape[1]
    nblk = N // NB

    def f(x_ref, ap_ref, cnt_ref, b_ref, u_ref, wx_ref, bn_ref, wg_ref,
          wu_ref, bg_ref, xo_ref, uo_ref, gacc, gcnt):
        i = pl.program_id(0)

        @pl.when(i == 0)
        def _():
            gacc[...] = jnp.zeros_like(gacc)
            gcnt[...] = jnp.zeros_like(gcnt)

        A = ap_ref[0] + ap_ref[1]
        c = cnt_ref[0][:, 0:1] + cnt_ref[1][:, 0:1]
        inv = 1.0 / jnp.maximum(c, 1.0)
        acc = jnp.dot(x_ref[...], wx_ref[...],
                      preferred_element_type=jnp.float32)
        acc += A * inv + bn_ref[...]
        xo = jnp.maximum(acc, 0.0)
        xo_ref[...] = xo
        b = b_ref[0]
        gids = lax.broadcasted_iota(jnp.int32, (G, NB), 0)
        S = (b == gids).astype(jnp.float32)
        gacc[...] += jnp.dot(S, xo, preferred_element_type=jnp.float32)
        gcnt[...] += jnp.broadcast_to(
            jnp.sum(S, axis=1, keepdims=True), gcnt.shape)

        @pl.when(i == nblk - 1)
        def _():
            gm = gacc[...] / jnp.maximum(gcnt[...], 1.0)
            uo = jnp.dot(gm, wg_ref[...], preferred_element_type=jnp.float32)
            uo += jnp.dot(u_ref[...], wu_ref[...],
                          preferred_element_type=jnp.float32)
            uo += bg_ref[...]
            uo_ref[...] = jnp.maximum(uo, 0.0)

    return pl.pallas_call(
        f,
        grid=(nblk,),
        in_specs=[pl.BlockSpec((NB, NF), lambda i: (i, 0)),
                  pl.BlockSpec((2, NB, NF), lambda i: (0, i, 0)),
                  pl.BlockSpec((2, NB, NF), lambda i: (0, i, 0)),
                  pl.BlockSpec((1, 1, NB), lambda i: (i, 0, 0)),
                  pl.BlockSpec((G, GU), lambda i: (0, 0)),
                  pl.BlockSpec((NF, NF), lambda i: (0, 0)),
                  pl.BlockSpec((1, NF), lambda i: (0, 0)),
                  pl.BlockSpec((NF, GO), lambda i: (0, 0)),
                  pl.BlockSpec((GU, GO), lambda i: (0, 0)),
                  pl.BlockSpec((1, GO), lambda i: (0, 0))],
        out_specs=[pl.BlockSpec((NB, NF), lambda i: (i, 0)),
                   pl.BlockSpec((G, GO), lambda i: (0, 0))],
        out_shape=[jax.ShapeDtypeStruct((N, NF), jnp.float32),
                   jax.ShapeDtypeStruct((G, GO), jnp.float32)],
        scratch_shapes=[pltpu.VMEM((G, NF), jnp.float32),
                        pltpu.VMEM((G, NF), jnp.float32)],
        compiler_params=pltpu.CompilerParams(
            dimension_semantics=("arbitrary",)),
    )(x, ap, cntp, batch3, u, Wn_x, bn.reshape(1, -1), Wg_g, Wg_u,
      bg.reshape(1, -1))


def _final(rgo, pgo, W1r, W1p, b1, W2pad, b2pad):
    def f(r_ref, p_ref, w1r_ref, w1p_ref, b1_ref, w2_ref, b2_ref, o_ref):
        y = jnp.dot(r_ref[...], w1r_ref[...],
                    preferred_element_type=jnp.float32)
        y += jnp.dot(p_ref[...], w1p_ref[...],
                     preferred_element_type=jnp.float32)
        y = jnp.maximum(y + b1_ref[...], 0.0)
        z = jnp.dot(y, w2_ref[...], preferred_element_type=jnp.float32)
        z += b2_ref[...]
        lane = lax.broadcasted_iota(jnp.int32, z.shape, 1)
        z = jnp.where(lane < OC, z, -1e30)
        m = jnp.max(z, axis=1, keepdims=True)
        s = jnp.sum(jnp.exp(z - m), axis=1, keepdims=True)
        o_ref[...] = z - m - jnp.log(s)

    return pl.pallas_call(
        f,
        out_shape=jax.ShapeDtypeStruct((G, 128), jnp.float32),
    )(rgo, pgo, W1r, W1p, b1.reshape(1, -1), W2pad, b2pad.reshape(1, -1))


class _GraphState:
    def __init__(self, x, e, ci, b, u, prefix):
        self.x, self.u, self.prefix = x, u, prefix
        self.src = jnp.pad(ci[0], (0, E_PAD - E))
        self.dst_g = jnp.pad(ci[1], (0, E_PAD - E))
        self.dst2d = jnp.pad(ci[1], (0, E_PAD - E),
                             constant_values=N).reshape(E_PAD // CH, CH)
        self.e = jnp.pad(e, ((0, E_PAD - E), (0, 0))).astype(jnp.bfloat16)
        self.batch3 = b.reshape(N // NB, 1, NB)


def _run_graphs(ga, gb, params):
    zeros = jnp.zeros((N_ACC, NF), jnp.float32)
    ones128 = jnp.ones((CH, NF), jnp.float32)
    for g in (ga, gb):
        g.cntp = _sc_count(g.dst2d, zeros, ones128).reshape(2, N, NF)
    for i in range(4):
        for g in (ga, gb):
            p = lambda n: params[g.prefix + str(i) + "_" + n]
            g.We, g.be = p("We"), p("be")
            g.Wn, g.bn = p("Wn"), p("bn")
            g.Wg, g.bg = p("Wg"), p("bg")
        for g in (ga, gb):
            g.xs, g.xd = _sc_gather2(g.x, g.src, g.dst_g)
        for g in (ga, gb):
            bf = jnp.bfloat16
            g.e, g.pm = _edge_layer(g.xs, g.xd, g.e, g.We[:NF].astype(bf),
                                    g.We[NF:2 * NF].astype(bf),
                                    g.We[2 * NF:].astype(bf), g.be,
                                    g.Wn[NF:], write_eout=(i < 3))
        for g in (ga, gb):
            g.ap = _sc_scatter(g.pm, g.dst2d, zeros).reshape(2, N, NF)
        for g in (ga, gb):
            g.x, g.u = _node_layer(g.x, g.ap, g.cntp, g.batch3, g.u,
                                   g.Wn[:NF], g.bn, g.Wg[:NF], g.Wg[NF:],
                                   g.bg)
    return ga.u, gb.u


def kernel(rx, re, rc, rb, rg, px, pe, pc, pb, pg, params):
    gr = _GraphState(rx, re, rc, rb, rg, "r")
    gp = _GraphState(px, pe, pc, pb, pg, "p")
    rgo, pgo = _run_graphs(gr, gp, params)
    W1, b1 = params["W1"], params["b1"]
    W2, b2 = params["W2"], params["b2"]
    W2pad = jnp.pad(W2, ((0, 0), (0, 128 - OC)))
    b2pad = jnp.pad(b2, (0, 128 - OC))
    out = _final(rgo, pgo, W1[:256], W1[256:], b1, W2pad, b2pad)
    return out[:, :OC]


# speedup vs baseline: 1.0737x; 1.0027x over previous
import functools

import jax
import jax.numpy as jnp
from jax import lax
from jax.experimental import pallas as pl
from jax.experimental.pallas import tpu as pltpu
from jax.experimental.pallas import tpu_sc as plsc

N = 10000
E = 160000
G = 64
NF = 128
EF = 16
NG = 16
OC = 32

NW = 32
CH = 128
E_PAD = 163840
CPW = E_PAD // (NW * CH)
N_ACC = N + CH
EB = 1280
NB = 2000

_mesh = lambda: plsc.VectorSubcoreMesh(core_axis_name="c", subcore_axis_name="s")


CPT = E_PAD // (16 * CH)
HALF = CPT // 2


def _sc_gather2x(xr, xp, sr, dr, sp, dp):
    @functools.partial(
        pl.kernel,
        mesh=_mesh(),
        out_type=[jax.ShapeDtypeStruct((E_PAD, NF), jnp.float32)] * 4,
        scratch_types=[pltpu.VMEM((HALF * CH,), jnp.int32),
                       pltpu.VMEM((HALF * CH,), jnp.int32),
                       pltpu.VMEM((CH, NF), jnp.float32),
                       pltpu.VMEM((CH, NF), jnp.float32),
                       pltpu.VMEM_SHARED((N, NF), jnp.float32),
                       pltpu.SemaphoreType.DMA,
                       pltpu.SemaphoreType.DMA],
    )
    def k(xr_hbm, xp_hbm, sr_hbm, dr_hbm, sp_hbm, dp_hbm,
          xsr_hbm, xdr_hbm, xsp_hbm, xdp_hbm,
          si_v, di_v, sb_v, db_v, x_sh, gsem, ssem):
        cid = lax.axis_index("c")
        sid = lax.axis_index("s")

        def flow(x_hbm, s_hbm, d_hbm, xs_hbm, xd_hbm):
            @pl.when(sid < 15)
            def _():
                pltpu.sync_copy(x_hbm.at[pl.ds(sid * 632, 632)],
                                x_sh.at[pl.ds(sid * 632, 632)])

            @pl.when(sid == 15)
            def _():
                pltpu.sync_copy(x_hbm.at[pl.ds(15 * 632, N - 15 * 632)],
                                x_sh.at[pl.ds(15 * 632, N - 15 * 632)])

            plsc.subcore_barrier()
            tb = sid * (CPT * CH)

            def half(h, _):
                hb = tb + h * (HALF * CH)
                pltpu.sync_copy(s_hbm.at[pl.ds(hb, HALF * CH)], si_v)
                pltpu.sync_copy(d_hbm.at[pl.ds(hb, HALF * CH)], di_v)

                def body(j, _):
                    off = j * CH
                    cs = pltpu.async_copy(x_sh.at[si_v.at[pl.ds(off, CH)]],
                                          sb_v, gsem)
                    cd = pltpu.async_copy(x_sh.at[di_v.at[pl.ds(off, CH)]],
                                          db_v, gsem)
                    cs.wait()
                    cd.wait()
                    s1 = pltpu.async_copy(sb_v,
                                          xs_hbm.at[pl.ds(hb + off, CH)],
                                          ssem)
                    s2 = pltpu.async_copy(db_v,
                                          xd_hbm.at[pl.ds(hb + off, CH)],
                                          ssem)
                    s1.wait()
                    s2.wait()
                    return 0

                lax.fori_loop(0, HALF, body, 0)
                return 0

            lax.fori_loop(0, 2, half, 0)

        @pl.when(cid == 0)
        def _():
            flow(xr_hbm, sr_hbm, dr_hbm, xsr_hbm, xdr_hbm)

        @pl.when(cid == 1)
        def _():
            flow(xp_hbm, sp_hbm, dp_hbm, xsp_hbm, xdp_hbm)

    return k(xr, xp, sr, dr, sp, dp)


def _sc_scatter2(pr, pp, dstr2d, dstp2d, zeros):
    ZR = 632
    WR = 624

    @functools.partial(
        pl.kernel,
        mesh=_mesh(),
        out_type=jax.ShapeDtypeStruct((2 * N, NF), jnp.float32),
        scratch_types=[pltpu.VMEM((CPT, CH), jnp.int32),
                       pltpu.VMEM((CH, NF), jnp.float32),
                       pltpu.VMEM((CH, NF), jnp.float32),
                       pltpu.VMEM_SHARED((N_ACC, NF), jnp.float32),
                       pltpu.SemaphoreType.DMA,
                       pltpu.SemaphoreType.DMA],
    )
    def k(pr_hbm, pp_hbm, dstr_hbm, dstp_hbm, z_hbm, out_hbm,
          idx_v, rows_a, rows_b, acc_sh, sem_a, sem_b):
        cid = lax.axis_index("c")
        sid = lax.axis_index("s")

        def flow(p_hbm, dst_hbm, ob):
            pltpu.sync_copy(z_hbm.at[pl.ds(sid * ZR, ZR)],
                            acc_sh.at[pl.ds(sid * ZR, ZR)])
            plsc.subcore_barrier()
            pltpu.sync_copy(dst_hbm.at[pl.ds(sid * CPT, CPT)], idx_v)
            eb = sid * CPT * CH
            pltpu.async_copy(p_hbm.at[pl.ds(eb, CH)], rows_a, sem_a)

            def body(gg, _):
                j0 = 2 * gg
                pltpu.async_copy(p_hbm.at[pl.ds(eb + (j0 + 1) * CH, CH)],
                                 rows_b, sem_b)
                pltpu.make_async_copy(p_hbm.at[pl.ds(eb, CH)], rows_a,
                                      sem_a).wait()
                pltpu.sync_copy(rows_a, acc_sh.at[idx_v.at[j0]], add=True)

                @pl.when(j0 + 2 < CPT)
                def _():
                    pltpu.async_copy(p_hbm.at[pl.ds(eb + (j0 + 2) * CH, CH)],
                                     rows_a, sem_a)

                pltpu.make_async_copy(p_hbm.at[pl.ds(eb, CH)], rows_b,
                                      sem_b).wait()
                pltpu.sync_copy(rows_b, acc_sh.at[idx_v.at[j0 + 1]],
                                add=True)
                return 0

            lax.fori_loop(0, CPT // 2, body, 0)
            plsc.subcore_barrier()
            pltpu.sync_copy(acc_sh.at[pl.ds(sid * WR, WR)],
                            out_hbm.at[pl.ds(ob + sid * WR, WR)])

            @pl.when(sid == 15)
            def _():
                pltpu.sync_copy(acc_sh.at[pl.ds(16 * WR, N - 16 * WR)],
                                out_hbm.at[pl.ds(ob + 16 * WR, N - 16 * WR)])

        @pl.when(cid == 0)
        def _():
            flow(pr_hbm, dstr_hbm, 0)

        @pl.when(cid == 1)
        def _():
            flow(pp_hbm, dstp_hbm, N)

    return k(pr, pp, dstr2d, dstp2d, zeros)


def _sc_count2(dstr2d, dstp2d, zeros, ones128):
    ZR = 632
    WR = 624

    @functools.partial(
        pl.kernel,
        mesh=_mesh(),
        out_type=jax.ShapeDtypeStruct((2 * N, NF), jnp.float32),
        scratch_types=[pltpu.VMEM((CPT, CH), jnp.int32),
                       pltpu.VMEM((CH, NF), jnp.float32),
                       pltpu.VMEM_SHARED((N_ACC, NF), jnp.float32)],
    )
    def k(dstr_hbm, dstp_hbm, z_hbm, ones_hbm, out_hbm, idx_v, ones_v,
          acc_sh):
        cid = lax.axis_index("c")
        sid = lax.axis_index("s")

        def flow(dst_hbm, ob):
            pltpu.sync_copy(z_hbm.at[pl.ds(sid * ZR, ZR)],
                            acc_sh.at[pl.ds(sid * ZR, ZR)])
            plsc.subcore_barrier()
            pltpu.sync_copy(ones_hbm, ones_v)
            pltpu.sync_copy(dst_hbm.at[pl.ds(sid * CPT, CPT)], idx_v)

            def body(j, _):
                pltpu.sync_copy(ones_v, acc_sh.at[idx_v.at[j]], add=True)
                return 0

            lax.fori_loop(0, CPT, body, 0)
            plsc.subcore_barrier()
            pltpu.sync_copy(acc_sh.at[pl.ds(sid * WR, WR)],
                            out_hbm.at[pl.ds(ob + sid * WR, WR)])

            @pl.when(sid == 15)
            def _():
                pltpu.sync_copy(acc_sh.at[pl.ds(16 * WR, N - 16 * WR)],
                                out_hbm.at[pl.ds(ob + 16 * WR, N - 16 * WR)])

        @pl.when(cid == 0)
        def _():
            flow(dstr_hbm, 0)

        @pl.when(cid == 1)
        def _():
            flow(dstp_hbm, N)

    return k(dstr2d, dstp2d, zeros, ones128)


def _edge_layer(xs, xd, e, We_s, We_d, We_e, be, Wn_a, write_eout):
    K = e.shape[1]
    EO = We_s.shape[1]
    grid = E_PAD // EB

    def f(xs_ref, xd_ref, e_ref, ws_ref, wd_ref, we_ref, be_ref, wa_ref,
          *out_refs):
        acc = jnp.dot(xs_ref[...].astype(jnp.bfloat16), ws_ref[...],
                      preferred_element_type=jnp.float32)
        acc += jnp.dot(xd_ref[...].astype(jnp.bfloat16), wd_ref[...],
                       preferred_element_type=jnp.float32)
        acc += jnp.dot(e_ref[...], we_ref[...],
                       preferred_element_type=jnp.float32)
        acc += be_ref[...]
        r = jnp.maximum(acc, 0.0)
        if write_eout:
            out_refs[1][...] = r.astype(jnp.bfloat16)
        out_refs[0][...] = jnp.dot(r, wa_ref[...],
                                   preferred_element_type=jnp.float32)

    out_shape = [jax.ShapeDtypeStruct((E_PAD, NF), jnp.float32)]
    out_specs = [pl.BlockSpec((EB, NF), lambda i: (i, 0))]
    if write_eout:
        out_shape.append(jax.ShapeDtypeStruct((E_PAD, EO), jnp.bfloat16))
        out_specs.append(pl.BlockSpec((EB, EO), lambda i: (i, 0)))

    res = pl.pallas_call(
        f,
        grid=(grid,),
        in_specs=[pl.BlockSpec((EB, NF), lambda i: (i, 0)),
                  pl.BlockSpec((EB, NF), lambda i: (i, 0)),
                  pl.BlockSpec((EB, K), lambda i: (i, 0)),
                  pl.BlockSpec((NF, EO), lambda i: (0, 0)),
                  pl.BlockSpec((NF, EO), lambda i: (0, 0)),
                  pl.BlockSpec((K, EO), lambda i: (0, 0)),
                  pl.BlockSpec((1, EO), lambda i: (0, 0)),
                  pl.BlockSpec((EO, NF), lambda i: (0, 0))],
        out_specs=out_specs,
        out_shape=out_shape,
    )(xs, xd, e, We_s, We_d, We_e, be.reshape(1, -1), Wn_a)
    if write_eout:
        return res[1], res[0]
    return None, res[0]


def _node_layer(x, ap, cntp, batch3, u, Wn_x, bn, Wg_g, Wg_u, bg, gb):
    GO = Wg_g.shape[1]
    GU = u.shape[1]
    nblk = N // NB

    def f(x_ref, ap_ref, cnt_ref, b_ref, u_ref, wx_ref, bn_ref, wg_ref,
          wu_ref, bg_ref, xo_ref, uo_ref, gacc, gcnt):
        i = pl.program_id(0)

        @pl.when(i == 0)
        def _():
            gacc[...] = jnp.zeros_like(gacc)
            gcnt[...] = jnp.zeros_like(gcnt)

        A = ap_ref[...]
        c = cnt_ref[:, 0:1]
        inv = 1.0 / jnp.maximum(c, 1.0)
        acc = jnp.dot(x_ref[...], wx_ref[...],
                      preferred_element_type=jnp.float32)
        acc += A * inv + bn_ref[...]
        xo = jnp.maximum(acc, 0.0)
        xo_ref[...] = xo
        b = b_ref[0]
        gids = lax.broadcasted_iota(jnp.int32, (G, NB), 0)
        S = (b == gids).astype(jnp.float32)
        gacc[...] += jnp.dot(S, xo, preferred_element_type=jnp.float32)
        gcnt[...] += jnp.broadcast_to(
            jnp.sum(S, axis=1, keepdims=True), gcnt.shape)

        @pl.when(i == nblk - 1)
        def _():
            gm = gacc[...] / jnp.maximum(gcnt[...], 1.0)
            uo = jnp.dot(gm, wg_ref[...], preferred_element_type=jnp.float32)
            uo += jnp.dot(u_ref[...], wu_ref[...],
                          preferred_element_type=jnp.float32)
            uo += bg_ref[...]
            uo_ref[...] = jnp.maximum(uo, 0.0)

    return pl.pallas_call(
        f,
        grid=(nblk,),
        in_specs=[pl.BlockSpec((NB, NF), lambda i: (i, 0)),
                  pl.BlockSpec((NB, NF), lambda i, gb=gb: (i + gb, 0)),
                  pl.BlockSpec((NB, NF), lambda i, gb=gb: (i + gb, 0)),
                  pl.BlockSpec((1, 1, NB), lambda i: (i, 0, 0)),
                  pl.BlockSpec((G, GU), lambda i: (0, 0)),
                  pl.BlockSpec((NF, NF), lambda i: (0, 0)),
                  pl.BlockSpec((1, NF), lambda i: (0, 0)),
                  pl.BlockSpec((NF, GO), lambda i: (0, 0)),
                  pl.BlockSpec((GU, GO), lambda i: (0, 0)),
                  pl.BlockSpec((1, GO), lambda i: (0, 0))],
        out_specs=[pl.BlockSpec((NB, NF), lambda i: (i, 0)),
                   pl.BlockSpec((G, GO), lambda i: (0, 0))],
        out_shape=[jax.ShapeDtypeStruct((N, NF), jnp.float32),
                   jax.ShapeDtypeStruct((G, GO), jnp.float32)],
        scratch_shapes=[pltpu.VMEM((G, NF), jnp.float32),
                        pltpu.VMEM((G, NF), jnp.float32)],
        compiler_params=pltpu.CompilerParams(
            dimension_semantics=("arbitrary",)),
    )(x, ap, cntp, batch3, u, Wn_x, bn.reshape(1, -1), Wg_g, Wg_u,
      bg.reshape(1, -1))


def _final(rgo, pgo, W1r, W1p, b1, W2pad, b2pad):
    def f(r_ref, p_ref, w1r_ref, w1p_ref, b1_ref, w2_ref, b2_ref, o_ref):
        y = jnp.dot(r_ref[...], w1r_ref[...],
                    preferred_element_type=jnp.float32)
        y += jnp.dot(p_ref[...], w1p_ref[...],
                     preferred_element_type=jnp.float32)
        y = jnp.maximum(y + b1_ref[...], 0.0)
        z = jnp.dot(y, w2_ref[...], preferred_element_type=jnp.float32)
        z += b2_ref[...]
        lane = lax.broadcasted_iota(jnp.int32, z.shape, 1)
        z = jnp.where(lane < OC, z, -1e30)
        m = jnp.max(z, axis=1, keepdims=True)
        s = jnp.sum(jnp.exp(z - m), axis=1, keepdims=True)
        o_ref[...] = z - m - jnp.log(s)

    return pl.pallas_call(
        f,
        out_shape=jax.ShapeDtypeStruct((G, 128), jnp.float32),
    )(rgo, pgo, W1r, W1p, b1.reshape(1, -1), W2pad, b2pad.reshape(1, -1))


class _GraphState:
    def __init__(self, x, e, ci, b, u, prefix):
        self.x, self.u, self.prefix = x, u, prefix
        self.src = jnp.pad(ci[0], (0, E_PAD - E))
        self.dst_g = jnp.pad(ci[1], (0, E_PAD - E))
        self.dst2d = jnp.pad(ci[1], (0, E_PAD - E),
                             constant_values=N).reshape(E_PAD // CH, CH)
        self.e = jnp.pad(e, ((0, E_PAD - E), (0, 0))).astype(jnp.bfloat16)
        self.batch3 = b.reshape(N // NB, 1, NB)


def _run_graphs(ga, gb, params):
    zeros = jnp.zeros((N_ACC, NF), jnp.float32)
    ones128 = jnp.ones((CH, NF), jnp.float32)
    cnt2 = _sc_count2(ga.dst2d, gb.dst2d, zeros, ones128)
    for i in range(4):
        for g in (ga, gb):
            p = lambda n: params[g.prefix + str(i) + "_" + n]
            g.We, g.be = p("We"), p("be")
            g.Wn, g.bn = p("Wn"), p("bn")
            g.Wg, g.bg = p("Wg"), p("bg")
        ga.xs, ga.xd, gb.xs, gb.xd = _sc_gather2x(
            ga.x, gb.x, ga.src, ga.dst_g, gb.src, gb.dst_g)
        for g in (ga, gb):
            bf = jnp.bfloat16
            g.e, g.pm = _edge_layer(g.xs, g.xd, g.e, g.We[:NF].astype(bf),
                                    g.We[NF:2 * NF].astype(bf),
                                    g.We[2 * NF:].astype(bf), g.be,
                                    g.Wn[NF:], write_eout=(i < 3))
        ap2 = _sc_scatter2(ga.pm, gb.pm, ga.dst2d, gb.dst2d, zeros)
        for g, goff in ((ga, 0), (gb, N // NB)):
            g.x, g.u = _node_layer(g.x, ap2, cnt2, g.batch3, g.u,
                                   g.Wn[:NF], g.bn, g.Wg[:NF], g.Wg[NF:],
                                   g.bg, gb=goff)
    return ga.u, gb.u


def kernel(rx, re, rc, rb, rg, px, pe, pc, pb, pg, params):
    gr = _GraphState(rx, re, rc, rb, rg, "r")
    gp = _GraphState(px, pe, pc, pb, pg, "p")
    rgo, pgo = _run_graphs(gr, gp, params)
    W1, b1 = params["W1"], params["b1"]
    W2, b2 = params["W2"], params["b2"]
    W2pad = jnp.pad(W2, ((0, 0), (0, 128 - OC)))
    b2pad = jnp.pad(b2, (0, 128 - OC))
    out = _final(rgo, pgo, W1[:256], W1[256:], b1, W2pad, b2pad)
    return out[:, :OC]
